# Initial kernel scaffold; baseline (speedup 1.0000x reference)
import jax
import jax.numpy as jnp
from jax.experimental import pallas as pl


def kernel(x, center):
    raise NotImplementedError("write your pallas kernel here")


# speedup vs baseline: 2.4476x; 2.4476x over previous
import functools

import jax
import jax.numpy as jnp
from jax import lax
from jax.experimental import pallas as pl
from jax.experimental.pallas import tpu as pltpu
from jax.experimental.pallas import tpu_sc as plsc

_BT = 512
_CH = 128


def _labels_tc(x, center_t):
    B, M, D = x.shape
    K = center_t.shape[2]

    def body(x_ref, ct_ref, lab_ref):
        cols = []
        for m in range(M):
            xm = x_ref[:, m, :]
            cm = ct_ref[m]
            csq = jnp.sum(cm * cm, axis=0, keepdims=True)
            xsq = jnp.sum(xm * xm, axis=1, keepdims=True)
            dot = lax.dot_general(
                xm, cm, (((1,), (0,)), ((), ())),
                preferred_element_type=jnp.float32)
            dist = (csq - 2.0 * dot) + xsq
            neg = -jnp.sqrt(dist)
            mx = jnp.max(neg, axis=1, keepdims=True)
            iota = lax.broadcasted_iota(jnp.int32, (_BT, K), 1)
            idx = jnp.min(jnp.where(neg == mx, iota, K), axis=1)
            idx = jnp.minimum(idx, K - 1)
            cols.append(idx[:, None])
        lab_ref[...] = jnp.concatenate(cols, axis=1)

    return pl.pallas_call(
        body,
        grid=(B // _BT,),
        in_specs=[
            pl.BlockSpec((_BT, M, D), lambda j: (j, 0, 0)),
            pl.BlockSpec((M, D, K), lambda j: (0, 0, 0)),
        ],
        out_specs=pl.BlockSpec((_BT, M), lambda j: (j, 0)),
        out_shape=jax.ShapeDtypeStruct((B, M), jnp.int32),
    )(x, center_t)


def _gather_sc(table, gidx3):
    NW, NCH, CH = gidx3.shape
    D = table.shape[1]
    info = plsc.get_sparse_core_info()
    NC = info.num_cores
    mesh = plsc.VectorSubcoreMesh(core_axis_name="c", subcore_axis_name="s")

    @functools.partial(
        pl.kernel,
        out_type=jax.ShapeDtypeStruct((NW * NCH * CH, D), jnp.float32),
        mesh=mesh,
        scratch_types=[
            pltpu.VMEM((NCH, CH), jnp.int32),
            pltpu.VMEM((CH, D), jnp.float32),
            pltpu.VMEM((CH, D), jnp.float32),
            pltpu.SemaphoreType.DMA,
            pltpu.SemaphoreType.DMA,
        ],
    )
    def gk(idx_hbm, table_hbm, out_hbm, idx_v, buf0, buf1, sem0, sem1):
        wid = lax.axis_index("s") * NC + lax.axis_index("c")
        base = wid * (NCH * CH)
        pltpu.sync_copy(idx_hbm.at[wid], idx_v)
        bufs = (buf0, buf1)
        sems = (sem0, sem1)
        cps = [pltpu.async_copy(table_hbm.at[idx_v.at[0]], buf0, sem0), None]
        for c in range(NCH):
            cur = c % 2
            nxt = (c + 1) % 2
            if c + 1 < NCH:
                cps[nxt] = pltpu.async_copy(
                    table_hbm.at[idx_v.at[c + 1]], bufs[nxt], sems[nxt])
            cps[cur].wait()
            pltpu.sync_copy(bufs[cur], out_hbm.at[pl.ds(base + c * CH, CH)])

    return gk(gidx3, table)


def kernel(x, center):
    B, M, D = x.shape
    K = center.shape[1]
    center_t = jnp.transpose(center, (0, 2, 1))
    labels = _labels_tc(x, center_t)

    info = plsc.get_sparse_core_info()
    NW = info.num_cores * info.num_subcores
    NCH = (B * M) // (NW * _CH)
    gidx = labels + (jnp.arange(M, dtype=jnp.int32) * K)[None, :]
    gidx3 = gidx.reshape(NW, NCH, _CH)
    out = _gather_sc(center.reshape(M * K, D), gidx3).reshape(B, M, D)
    return (out, center, labels[..., None])


# speedup vs baseline: 3.4380x; 1.4047x over previous
import functools

import jax
import jax.numpy as jnp
from jax import lax
from jax.experimental import pallas as pl
from jax.experimental.pallas import tpu as pltpu
from jax.experimental.pallas import tpu_sc as plsc

_BT = 512
_CH = 128


def _labels_tc(x, center):
    B, M, D = x.shape
    K = center.shape[1]

    def body(x_ref, c_ref, lab_ref):
        ones_dc = jnp.ones((D, 128), jnp.float32)
        ones_rd = jnp.ones((8, D), jnp.float32)
        kio = lax.broadcasted_iota(jnp.int32, (K, _BT), 0).astype(jnp.float32)
        rows = []
        for m in range(M):
            xm = x_ref[:, m, :]
            cm = c_ref[m]
            csq = lax.dot_general(
                cm * cm, ones_dc, (((1,), (0,)), ((), ())),
                preferred_element_type=jnp.float32)[:, :1]
            xsq = lax.dot_general(
                ones_rd, xm * xm, (((1,), (1,)), ((), ())),
                preferred_element_type=jnp.float32)[:1, :]
            dot = lax.dot_general(
                cm, xm, (((1,), (1,)), ((), ())),
                preferred_element_type=jnp.float32)
            dist = (csq - 2.0 * dot) + xsq
            mn = jnp.min(dist, axis=0, keepdims=True)
            s1 = jnp.sqrt(mn)
            hi = mn
            for j in (1, 2, 3):
                u = lax.bitcast_convert_type(
                    lax.bitcast_convert_type(mn, jnp.int32) + j, jnp.float32)
                hi = jnp.where(jnp.sqrt(u) == s1, u, hi)
            sel = jnp.where(dist <= hi, kio, float(K))
            rows.append(jnp.min(sel, axis=0, keepdims=True))
        lab = jnp.concatenate(rows, axis=0).astype(jnp.int32)
        lab_ref[...] = jnp.minimum(lab, K - 1)

    return pl.pallas_call(
        body,
        grid=(B // _BT,),
        in_specs=[
            pl.BlockSpec((_BT, M, D), lambda j: (j, 0, 0)),
            pl.BlockSpec((M, K, D), lambda j: (0, 0, 0)),
        ],
        out_specs=pl.BlockSpec((M, _BT), lambda j: (0, j)),
        out_shape=jax.ShapeDtypeStruct((M, B), jnp.int32),
    )(x, center)


def _gather_sc(table, gidx3):
    NW, NCH, CH = gidx3.shape
    D = table.shape[1]
    info = plsc.get_sparse_core_info()
    NC = info.num_cores
    mesh = plsc.VectorSubcoreMesh(core_axis_name="c", subcore_axis_name="s")

    @functools.partial(
        pl.kernel,
        out_type=jax.ShapeDtypeStruct((NW * NCH * CH, D), jnp.float32),
        mesh=mesh,
        scratch_types=[
            pltpu.VMEM((NCH, CH), jnp.int32),
            pltpu.VMEM((CH, D), jnp.float32),
            pltpu.VMEM((CH, D), jnp.float32),
            pltpu.SemaphoreType.DMA,
            pltpu.SemaphoreType.DMA,
        ],
    )
    def gk(idx_hbm, table_hbm, out_hbm, idx_v, buf0, buf1, sem0, sem1):
        wid = lax.axis_index("s") * NC + lax.axis_index("c")
        base = wid * (NCH * CH)
        pltpu.sync_copy(idx_hbm.at[wid], idx_v)
        bufs = (buf0, buf1)
        sems = (sem0, sem1)
        cps = [pltpu.async_copy(table_hbm.at[idx_v.at[0]], buf0, sem0), None]
        for c in range(NCH):
            cur = c % 2
            nxt = (c + 1) % 2
            if c + 1 < NCH:
                cps[nxt] = pltpu.async_copy(
                    table_hbm.at[idx_v.at[c + 1]], bufs[nxt], sems[nxt])
            cps[cur].wait()
            pltpu.sync_copy(bufs[cur], out_hbm.at[pl.ds(base + c * CH, CH)])

    return gk(gidx3, table)


def kernel(x, center):
    B, M, D = x.shape
    K = center.shape[1]
    labels_mb = _labels_tc(x, center)
    labels = labels_mb.T

    info = plsc.get_sparse_core_info()
    NW = info.num_cores * info.num_subcores
    NCH = (B * M) // (NW * _CH)
    gidx = labels + (jnp.arange(M, dtype=jnp.int32) * K)[None, :]
    gidx3 = gidx.reshape(NW, NCH, _CH)
    out = _gather_sc(center.reshape(M * K, D), gidx3).reshape(B, M, D)
    return (out, center, labels[..., None])
